# baseline (speedup 1.0000x reference)
import jax
import jax.numpy as jnp
from jax import lax
from jax.experimental import pallas as pl
from jax.experimental.pallas import tpu as pltpu
from jax.experimental.pallas import tpu_sc as plsc

_MEM_ROWS = 65536
_ASSETS = 512
_NUM_WORKERS = 32
_WROWS = _MEM_ROWS // _NUM_WORKERS
_R = 64
_NCHUNKS = _WROWS // _R
_NBUF = 3
_LA = 1


def _shift_body(new_hbm, mem_hbm, out_hbm, *scratch):
    cid = lax.axis_index("c")
    sid = lax.axis_index("s")
    wid = sid * 2 + cid
    base = wid * _WROWS

    bufs = scratch[:_NBUF]
    isems = scratch[_NBUF:2 * _NBUF]
    osems = scratch[2 * _NBUF:3 * _NBUF]

    last = _NCHUNKS - 1

    def gather_start(c):
        b = c % _NBUF
        o = base + c * _R
        if c == last:
            @pl.when(wid == _NUM_WORKERS - 1)
            def _tail():
                pltpu.make_async_copy(
                    mem_hbm.at[pl.ds(o + 1, _R - 1)],
                    bufs[b].at[pl.ds(0, _R - 1)], isems[b]).start()
                pltpu.make_async_copy(new_hbm, bufs[b].at[_R - 1],
                                      isems[b]).start()

            @pl.when(wid < _NUM_WORKERS - 1)
            def _bulk():
                pltpu.make_async_copy(
                    mem_hbm.at[pl.ds(o + 1, _R)], bufs[b], isems[b]).start()
        else:
            pltpu.make_async_copy(
                mem_hbm.at[pl.ds(o + 1, _R)], bufs[b], isems[b]).start()

    def gather_wait(c):
        b = c % _NBUF
        pltpu.make_async_copy(
            mem_hbm.at[pl.ds(0, _R)], bufs[b], isems[b]).wait()

    def scatter(c):
        b = c % _NBUF
        o = base + c * _R
        return pltpu.make_async_copy(
            bufs[b], out_hbm.at[pl.ds(o, _R)], osems[b])

    for c in range(_NBUF):
        gather_start(c)

    for c in range(_NCHUNKS):
        gather_wait(c)
        scatter(c).start()
        k = c + _LA
        if _NBUF <= k < _NCHUNKS:
            scatter(k - _NBUF).wait()
            gather_start(k)

    for c in range(_NCHUNKS - _NBUF, _NCHUNKS):
        scatter(c).wait()


@jax.jit
def _shift(new, memory):
    mesh = plsc.VectorSubcoreMesh(core_axis_name="c", subcore_axis_name="s")
    return pl.kernel(
        _shift_body,
        out_type=jax.ShapeDtypeStruct((_MEM_ROWS, _ASSETS), jnp.float32),
        mesh=mesh,
        compiler_params=pltpu.CompilerParams(use_tc_tiling_on_sc=False),
        scratch_types=(
            [pltpu.VMEM((_R, _ASSETS), jnp.float32)] * _NBUF
            + [pltpu.SemaphoreType.DMA] * (2 * _NBUF)
        ),
    )(new, memory)


def kernel(new, memory):
    return _shift(new, memory)


# speedup vs baseline: 1.3657x; 1.3657x over previous
import jax
import jax.numpy as jnp
from jax import lax
from jax.experimental import pallas as pl
from jax.experimental.pallas import tpu as pltpu
from jax.experimental.pallas import tpu_sc as plsc

_MEM_ROWS = 65536
_ASSETS = 512
_NUM_WORKERS = 32
_WROWS = _MEM_ROWS // _NUM_WORKERS
_R = 32
_R8 = _R + 8
_N = _WROWS // _R
_NBUF = 3
_NLANE = _ASSETS // 16


def _shift_body(new_hbm, mem_hbm, out_hbm, *scratch):
    cid = lax.axis_index("c")
    sid = lax.axis_index("s")
    wid = sid * 2 + cid
    base = wid * _WROWS

    ibufs = scratch[0:_NBUF]
    obufs = scratch[_NBUF:2 * _NBUF]
    isems = scratch[2 * _NBUF:3 * _NBUF]
    osems = scratch[3 * _NBUF:4 * _NBUF]
    newbuf = scratch[4 * _NBUF]
    nsem = scratch[4 * _NBUF + 1]

    w_last = _NUM_WORKERS - 1

    new_cp = pltpu.make_async_copy(new_hbm, newbuf, nsem)

    @pl.when(wid == w_last)
    def _stage_new():
        new_cp.start()

    def start_g(c, b):
        o = base + c * _R
        pltpu.make_async_copy(
            mem_hbm.at[pl.ds(o, _R8)], ibufs[b], isems[b]).start()

    def start_g_short(c, b):
        o = base + c * _R
        pltpu.make_async_copy(
            mem_hbm.at[pl.ds(o, _R)], ibufs[b].at[pl.ds(0, _R)],
            isems[b]).start()

    def wait_g(b):
        pltpu.make_async_copy(
            mem_hbm.at[pl.ds(0, _R8)], ibufs[b], isems[b]).wait()

    def wait_g_short(b):
        pltpu.make_async_copy(
            mem_hbm.at[pl.ds(0, _R)], ibufs[b].at[pl.ds(0, _R)],
            isems[b]).wait()

    def rotate(b, nrows=_R):
        def row(j, carry):
            for cs in range(_NLANE):
                obufs[b][j, pl.ds(cs * 16, 16)] = (
                    ibufs[b][j + 1, pl.ds(cs * 16, 16)])
            return carry
        lax.fori_loop(0, nrows, row, 0)

    def start_s(c, b):
        o = base + c * _R
        pltpu.make_async_copy(
            obufs[b], out_hbm.at[pl.ds(o, _R)], osems[b]).start()

    def wait_s(b):
        pltpu.make_async_copy(
            obufs[b], out_hbm.at[pl.ds(0, _R)], osems[b]).wait()

    start_g(0, 0)

    start_g(1, 1)
    wait_g(0)
    rotate(0)
    start_s(0, 0)

    start_g(2, 2)
    wait_g(1)
    rotate(1)
    start_s(1, 1)

    wait_s(0)
    start_g(3, 0)
    wait_g(2)
    rotate(2)
    start_s(2, 2)

    def group(g, carry):
        for jj in range(3):
            c = 3 * g + jj
            b = jj
            nb = (jj + 1) % 3
            wait_s(nb)
            start_g(c + 1, nb)
            wait_g(b)
            rotate(b)
            start_s(c, b)
        return carry

    lax.fori_loop(1, 20, group, 0)

    wait_s(1)
    start_g(61, 1)
    wait_g(0)
    rotate(0)
    start_s(60, 0)

    wait_s(2)
    start_g(62, 2)
    wait_g(1)
    rotate(1)
    start_s(61, 1)

    wait_s(0)

    @pl.when(wid == w_last)
    def _g63_tail():
        start_g_short(63, 0)

    @pl.when(wid < w_last)
    def _g63_bulk():
        start_g(63, 0)

    wait_g(2)
    rotate(2)
    start_s(62, 2)

    @pl.when(wid == w_last)
    def _c63_tail():
        wait_g_short(0)
        new_cp.wait()
        rotate(0, nrows=_R - 1)
        for cs in range(_NLANE):
            obufs[0][_R - 1, pl.ds(cs * 16, 16)] = newbuf[pl.ds(cs * 16, 16)]

    @pl.when(wid < w_last)
    def _c63_bulk():
        wait_g(0)
        rotate(0)

    start_s(63, 0)

    wait_s(1)
    wait_s(2)
    wait_s(0)


@jax.jit
def _shift(new, memory):
    mesh = plsc.VectorSubcoreMesh(core_axis_name="c", subcore_axis_name="s")
    return pl.kernel(
        _shift_body,
        out_type=jax.ShapeDtypeStruct((_MEM_ROWS, _ASSETS), jnp.float32),
        mesh=mesh,
        scratch_types=(
            [pltpu.VMEM((_R8, _ASSETS), jnp.float32)] * _NBUF
            + [pltpu.VMEM((_R, _ASSETS), jnp.float32)] * _NBUF
            + [pltpu.SemaphoreType.DMA] * (2 * _NBUF)
            + [pltpu.VMEM((_ASSETS,), jnp.float32), pltpu.SemaphoreType.DMA]
        ),
    )(new, memory)


def kernel(new, memory):
    return _shift(new, memory)


# speedup vs baseline: 1.3695x; 1.0028x over previous
import jax
import jax.numpy as jnp
from jax import lax
from jax.experimental import pallas as pl
from jax.experimental.pallas import tpu as pltpu
from jax.experimental.pallas import tpu_sc as plsc

_MEM_ROWS = 65536
_ASSETS = 512
_NUM_WORKERS = 32
_WROWS = _MEM_ROWS // _NUM_WORKERS
_R = 32
_R8 = _R + 8
_N = _WROWS // _R
_NBUF = 3
_NLANE = _ASSETS // 16


def _shift_body(new_hbm, mem_hbm, out_hbm, ibuf, obuf, newbuf,
                isem0, isem1, isem2, osem0, osem1, osem2, nsem):
    cid = lax.axis_index("c")
    sid = lax.axis_index("s")
    wid = sid * 2 + cid
    base = wid * _WROWS

    isems = (isem0, isem1, isem2)
    osems = (osem0, osem1, osem2)
    w_last = _NUM_WORKERS - 1

    new_cp = pltpu.make_async_copy(new_hbm, newbuf, nsem)

    @pl.when(wid == w_last)
    def _stage_new():
        new_cp.start()

    def for_buf(b, fn):
        for i in range(_NBUF):
            @pl.when(b == i)
            def _(i=i):
                fn(i)

    def start_g(c, b):
        o = base + c * _R
        for_buf(b, lambda i: pltpu.make_async_copy(
            mem_hbm.at[pl.ds(o, _R8)], ibuf.at[i], isems[i]).start())

    def start_g_short(c, b):
        o = base + c * _R
        for_buf(b, lambda i: pltpu.make_async_copy(
            mem_hbm.at[pl.ds(o, _R)], ibuf.at[i, pl.ds(0, _R)],
            isems[i]).start())

    def wait_g(b):
        for_buf(b, lambda i: pltpu.make_async_copy(
            mem_hbm.at[pl.ds(0, _R8)], ibuf.at[i], isems[i]).wait())

    def wait_g_short(b):
        for_buf(b, lambda i: pltpu.make_async_copy(
            mem_hbm.at[pl.ds(0, _R)], ibuf.at[i, pl.ds(0, _R)],
            isems[i]).wait())

    def start_s(c, b):
        o = base + c * _R
        for_buf(b, lambda i: pltpu.make_async_copy(
            obuf.at[i], out_hbm.at[pl.ds(o, _R)], osems[i]).start())

    def wait_s(b):
        for_buf(b, lambda i: pltpu.make_async_copy(
            obuf.at[i], out_hbm.at[pl.ds(0, _R)], osems[i]).wait())

    def rotate(b):
        def group(g, carry):
            r0 = 8 * g
            for r in range(8):
                for cs in range(_NLANE):
                    col = cs * 16
                    obuf[b, r0 + r, pl.ds(col, 16)] = (
                        ibuf[b, r0 + r + 1, pl.ds(col, 16)])
            return carry
        lax.fori_loop(0, _R // 8, group, 0)

    def chunk_body(c, carry):
        b = lax.rem(c, _NBUF)
        nb = lax.rem(c + 1, _NBUF)
        is_tail_next = jnp.logical_and(c + 1 == _N - 1, wid == w_last)
        is_tail = jnp.logical_and(c == _N - 1, wid == w_last)

        @pl.when(c + 1 < _N)
        def _lookahead():
            @pl.when(c >= _NBUF - 1)
            def _():
                wait_s(nb)

            @pl.when(is_tail_next)
            def _():
                start_g_short(c + 1, nb)

            @pl.when(jnp.logical_not(is_tail_next))
            def _():
                start_g(c + 1, nb)

        @pl.when(is_tail)
        def _():
            wait_g_short(b)

        @pl.when(jnp.logical_not(is_tail))
        def _():
            wait_g(b)

        rotate(b)

        @pl.when(is_tail)
        def _fixup():
            new_cp.wait()
            for_buf(b, _write_new)

        start_s(c, b)
        return carry

    def _write_new(i):
        for cs in range(_NLANE):
            col = cs * 16
            obuf[i, _R - 1, pl.ds(col, 16)] = newbuf[pl.ds(col, 16)]

    start_g(0, 0)
    lax.fori_loop(0, _N, chunk_body, 0)

    for c in range(_N - _NBUF, _N):
        i = c % _NBUF
        pltpu.make_async_copy(
            obuf.at[i], out_hbm.at[pl.ds(0, _R)], osems[i]).wait()


@jax.jit
def _shift(new, memory):
    mesh = plsc.VectorSubcoreMesh(core_axis_name="c", subcore_axis_name="s")
    return pl.kernel(
        _shift_body,
        out_type=jax.ShapeDtypeStruct((_MEM_ROWS, _ASSETS), jnp.float32),
        mesh=mesh,
        scratch_types=(
            [pltpu.VMEM((_NBUF, _R8, _ASSETS), jnp.float32),
             pltpu.VMEM((_NBUF, _R, _ASSETS), jnp.float32),
             pltpu.VMEM((_ASSETS,), jnp.float32)]
            + [pltpu.SemaphoreType.DMA] * (2 * _NBUF + 1)
        ),
    )(new, memory)


def kernel(new, memory):
    return _shift(new, memory)


# speedup vs baseline: 2.8040x; 2.0475x over previous
import jax
import jax.numpy as jnp
from jax import lax
from jax.experimental import pallas as pl
from jax.experimental.pallas import tpu as pltpu
from jax.experimental.pallas import tpu_sc as plsc

_MEM_ROWS = 65536
_ASSETS = 512
_NUM_WORKERS = 32
_WROWS = _MEM_ROWS // _NUM_WORKERS
_R = 32
_R8 = _R + 8
_N = _WROWS // _R
_NBUF = 4
_LA = 2
_NLANE = _ASSETS // 16


def _shift_body(new_hbm, mem_hbm, out_hbm, buf, newbuf,
                isem0, isem1, isem2, isem3, osem0, osem1, osem2, osem3,
                nsem):
    cid = lax.axis_index("c")
    sid = lax.axis_index("s")
    wid = sid * 2 + cid
    base = wid * _WROWS

    isems = (isem0, isem1, isem2, isem3)
    osems = (osem0, osem1, osem2, osem3)
    w_last = _NUM_WORKERS - 1

    new_cp = pltpu.make_async_copy(new_hbm, newbuf, nsem)

    @pl.when(wid == w_last)
    def _stage_new():
        new_cp.start()

    def for_buf(b, fn):
        for i in range(_NBUF):
            @pl.when(b == i)
            def _(i=i):
                fn(i)

    def start_g(c, b):
        o = base + c * _R
        for_buf(b, lambda i: pltpu.make_async_copy(
            mem_hbm.at[pl.ds(o, _R8)], buf.at[i], isems[i]).start())

    def start_g_short(c, b):
        o = base + c * _R
        for_buf(b, lambda i: pltpu.make_async_copy(
            mem_hbm.at[pl.ds(o, _R)], buf.at[i, pl.ds(0, _R)],
            isems[i]).start())

    def wait_g(b):
        for_buf(b, lambda i: pltpu.make_async_copy(
            mem_hbm.at[pl.ds(0, _R8)], buf.at[i], isems[i]).wait())

    def wait_g_short(b):
        for_buf(b, lambda i: pltpu.make_async_copy(
            mem_hbm.at[pl.ds(0, _R)], buf.at[i, pl.ds(0, _R)],
            isems[i]).wait())

    def start_s(c, b):
        o = base + c * _R
        for_buf(b, lambda i: pltpu.make_async_copy(
            buf.at[i, pl.ds(0, _R)], out_hbm.at[pl.ds(o, _R)],
            osems[i]).start())

    def wait_s(b):
        for_buf(b, lambda i: pltpu.make_async_copy(
            buf.at[i, pl.ds(0, _R)], out_hbm.at[pl.ds(0, _R)],
            osems[i]).wait())

    def rotate(b):
        def group(g, carry):
            r0 = 8 * g
            for r in range(8):
                for cs in range(_NLANE):
                    col = cs * 16
                    buf[b, r0 + r, pl.ds(col, 16)] = (
                        buf[b, r0 + r + 1, pl.ds(col, 16)])
            return carry
        lax.fori_loop(0, _R // 8, group, 0)

    def _write_new(i):
        for cs in range(_NLANE):
            col = cs * 16
            buf[i, _R - 1, pl.ds(col, 16)] = newbuf[pl.ds(col, 16)]

    def chunk_body(c, carry):
        b = lax.rem(c, _NBUF)
        kb = lax.rem(c + _LA, _NBUF)
        is_tail_next = jnp.logical_and(c + _LA == _N - 1, wid == w_last)
        is_tail = jnp.logical_and(c == _N - 1, wid == w_last)

        @pl.when(c + _LA < _N)
        def _lookahead():
            @pl.when(c >= _NBUF - _LA)
            def _():
                wait_s(kb)

            @pl.when(is_tail_next)
            def _():
                start_g_short(c + _LA, kb)

            @pl.when(jnp.logical_not(is_tail_next))
            def _():
                start_g(c + _LA, kb)

        @pl.when(is_tail)
        def _():
            wait_g_short(b)

        @pl.when(jnp.logical_not(is_tail))
        def _():
            wait_g(b)

        rotate(b)

        @pl.when(is_tail)
        def _fixup():
            new_cp.wait()
            for_buf(b, _write_new)

        start_s(c, b)
        return carry

    start_g(0, 0)
    start_g(1, 1)
    lax.fori_loop(0, _N, chunk_body, 0)

    for c in range(_N - _NBUF, _N):
        i = c % _NBUF
        pltpu.make_async_copy(
            buf.at[i, pl.ds(0, _R)], out_hbm.at[pl.ds(0, _R)],
            osems[i]).wait()


@jax.jit
def _shift(new, memory):
    mesh = plsc.VectorSubcoreMesh(core_axis_name="c", subcore_axis_name="s")
    return pl.kernel(
        _shift_body,
        out_type=jax.ShapeDtypeStruct((_MEM_ROWS, _ASSETS), jnp.float32),
        mesh=mesh,
        scratch_types=(
            [pltpu.VMEM((_NBUF, _R8, _ASSETS), jnp.float32),
             pltpu.VMEM((_ASSETS,), jnp.float32)]
            + [pltpu.SemaphoreType.DMA] * (2 * _NBUF + 1)
        ),
    )(new, memory)


def kernel(new, memory):
    return _shift(new, memory)


# speedup vs baseline: 3.0205x; 1.0772x over previous
import jax
import jax.numpy as jnp
from jax import lax
from jax.experimental import pallas as pl
from jax.experimental.pallas import tpu as pltpu
from jax.experimental.pallas import tpu_sc as plsc

_MEM_ROWS = 65536
_ASSETS = 512
_NUM_WORKERS = 32
_WROWS = _MEM_ROWS // _NUM_WORKERS
_R = 32
_N = _WROWS // _R
_NBUF = 6
_LA = 3
_NLANE = _ASSETS // 16


def _shift_body(new_hbm, mem_hbm, out_hbm, buf, bbuf, newbuf,
                isem0, isem1, isem2, isem3, isem4, isem5,
                osem0, osem1, osem2, osem3, osem4, osem5,
                bsem, nsem):
    cid = lax.axis_index("c")
    sid = lax.axis_index("s")
    wid = sid * 2 + cid
    base = wid * _WROWS

    isems = (isem0, isem1, isem2, isem3, isem4, isem5)
    osems = (osem0, osem1, osem2, osem3, osem4, osem5)
    w_last = _NUM_WORKERS - 1

    new_cp = pltpu.make_async_copy(new_hbm, newbuf, nsem)
    bb_cp = pltpu.make_async_copy(
        mem_hbm.at[pl.ds(jnp.minimum(base + _WROWS, _MEM_ROWS - 8), 8)],
        bbuf, bsem)

    @pl.when(wid == w_last)
    def _stage_new():
        new_cp.start()

    @pl.when(wid < w_last)
    def _stage_boundary():
        bb_cp.start()

    def for_buf(b, fn):
        for i in range(_NBUF):
            @pl.when(b == i)
            def _(i=i):
                fn(i)

    def start_g(c, b):
        o = base + c * _R
        for_buf(b, lambda i: pltpu.make_async_copy(
            mem_hbm.at[pl.ds(o, _R)], buf.at[i], isems[i]).start())

    def wait_g(b):
        for_buf(b, lambda i: pltpu.make_async_copy(
            mem_hbm.at[pl.ds(0, _R)], buf.at[i], isems[i]).wait())

    def start_s(c, b):
        o = base + c * _R
        for_buf(b, lambda i: pltpu.make_async_copy(
            buf.at[i], out_hbm.at[pl.ds(o, _R)], osems[i]).start())

    def wait_s(b):
        for_buf(b, lambda i: pltpu.make_async_copy(
            buf.at[i], out_hbm.at[pl.ds(0, _R)], osems[i]).wait())

    def rotate(b):
        def group(g, carry):
            r0 = 8 * g
            for r in range(8):
                for cs in range(_NLANE):
                    col = cs * 16
                    buf[b, r0 + r, pl.ds(col, 16)] = (
                        buf[b, r0 + r + 1, pl.ds(col, 16)])
            return carry
        lax.fori_loop(0, _R // 8 - 1, group, 0)
        for r in range(24, _R - 1):
            for cs in range(_NLANE):
                col = cs * 16
                buf[b, r, pl.ds(col, 16)] = buf[b, r + 1, pl.ds(col, 16)]

    def chunk_body(c, carry):
        b = lax.rem(c, _NBUF)
        b1 = lax.rem(c + 1, _NBUF)
        kb = lax.rem(c + _LA, _NBUF)

        @pl.when(c + _LA < _N)
        def _lookahead():
            @pl.when(c >= _NBUF - _LA)
            def _():
                wait_s(kb)
            start_g(c + _LA, kb)

        @pl.when(c < _N - 1)
        def _():
            wait_g(b1)

        rotate(b)

        @pl.when(c < _N - 1)
        def _row31_next():
            def mv(i):
                j = (i + 1) % _NBUF
                for cs in range(_NLANE):
                    col = cs * 16
                    buf[i, _R - 1, pl.ds(col, 16)] = buf[j, 0, pl.ds(col, 16)]
            for_buf(b, mv)

        @pl.when(c == _N - 1)
        def _row31_tail():
            @pl.when(wid < w_last)
            def _():
                bb_cp.wait()
                for_buf(b, lambda i: _copy_row(i, bbuf))

            @pl.when(wid == w_last)
            def _():
                new_cp.wait()
                for_buf(b, lambda i: _copy_row_flat(i, newbuf))

        start_s(c, b)
        return carry

    def _copy_row(i, src):
        for cs in range(_NLANE):
            col = cs * 16
            buf[i, _R - 1, pl.ds(col, 16)] = src[0, pl.ds(col, 16)]

    def _copy_row_flat(i, src):
        for cs in range(_NLANE):
            col = cs * 16
            buf[i, _R - 1, pl.ds(col, 16)] = src[pl.ds(col, 16)]

    for c in range(_LA):
        start_g(c, c)
    wait_g(0)
    lax.fori_loop(0, _N, chunk_body, 0)

    for c in range(_N - _NBUF, _N):
        i = c % _NBUF
        pltpu.make_async_copy(
            buf.at[i], out_hbm.at[pl.ds(0, _R)], osems[i]).wait()


@jax.jit
def _shift(new, memory):
    mesh = plsc.VectorSubcoreMesh(core_axis_name="c", subcore_axis_name="s")
    return pl.kernel(
        _shift_body,
        out_type=jax.ShapeDtypeStruct((_MEM_ROWS, _ASSETS), jnp.float32),
        mesh=mesh,
        scratch_types=(
            [pltpu.VMEM((_NBUF, _R, _ASSETS), jnp.float32),
             pltpu.VMEM((8, _ASSETS), jnp.float32),
             pltpu.VMEM((_ASSETS,), jnp.float32)]
            + [pltpu.SemaphoreType.DMA] * (2 * _NBUF + 2)
        ),
    )(new, memory)


def kernel(new, memory):
    return _shift(new, memory)
